# baseline (speedup 1.0000x reference)
import functools

import jax
import jax.numpy as jnp
from jax import lax
from jax.experimental import pallas as pl
from jax.experimental.pallas import tpu as pltpu
from jax.experimental.pallas import tpu_sc as plsc

NC = 2
NS = 16
LANES = 16
NW = NC * NS

B = 4
S = 8192
D = 128
BT = B * S
PER_W = BT // NW
CHUNK = 128
NCH = PER_W // CHUNK

_mesh = plsc.VectorSubcoreMesh(
    core_axis_name="c", subcore_axis_name="s", num_cores=NC, num_subcores=NS
)


@functools.partial(
    pl.kernel,
    out_type=jax.ShapeDtypeStruct((BT, D), jnp.float32),
    mesh=_mesh,
    scratch_types=[
        pltpu.VMEM((NCH, CHUNK), jnp.int32),
        pltpu.VMEM((CHUNK, D), jnp.float32),
        pltpu.VMEM((CHUNK, D), jnp.float32),
        pltpu.SemaphoreType.DMA,
    ],
)
def _emb(x2_hbm, tab_hbm, pe_hbm, out_hbm, idx_v, rows_v, pe_v, sem):
    wid = lax.axis_index("s") * NC + lax.axis_index("c")
    base = wid * PER_W
    pos_base = lax.rem(base, S)
    pltpu.sync_copy(x2_hbm.at[pl.ds(wid * NCH, NCH)], idx_v)
    for j in range(NCH):
        pltpu.async_copy(tab_hbm.at[idx_v.at[j]], rows_v, sem).wait()
        pltpu.sync_copy(pe_hbm.at[pl.ds(pos_base + j * CHUNK, CHUNK)], pe_v)

        def row_body(i, _):
            for k in range(D // LANES):
                sl = pl.ds(k * LANES, LANES)
                rows_v[i, sl] = rows_v[i, sl] + pe_v[i, sl]
            return 0

        lax.fori_loop(0, CHUNK, row_body, 0)
        pltpu.sync_copy(rows_v, out_hbm.at[pl.ds(base + j * CHUNK, CHUNK)])


def kernel(x, tok_table, pe):
    x2 = x.reshape(BT // CHUNK, CHUNK)
    out = _emb(x2, tok_table, pe)
    return out.reshape(B, S, D)


# speedup vs baseline: 1.4398x; 1.4398x over previous
import functools

import jax
import jax.numpy as jnp
from jax import lax
from jax.experimental import pallas as pl
from jax.experimental.pallas import tpu as pltpu
from jax.experimental.pallas import tpu_sc as plsc

NC = 2
NS = 16
LANES = 16
NW = NC * NS

B = 4
S = 8192
D = 128
BT = B * S
POS_W = S // NW
CHUNK = 128
NCHUNK = B * (POS_W // CHUNK)
NBUF = 4
LOOKAHEAD = 3

_mesh = plsc.VectorSubcoreMesh(
    core_axis_name="c", subcore_axis_name="s", num_cores=NC, num_subcores=NS
)


@functools.partial(
    pl.kernel,
    out_type=jax.ShapeDtypeStruct((BT, D), jnp.float32),
    mesh=_mesh,
    scratch_types=[
        pltpu.VMEM((NCHUNK, CHUNK), jnp.int32),
        pltpu.VMEM((POS_W, D), jnp.float32),
        [pltpu.VMEM((CHUNK, D), jnp.float32) for _ in range(NBUF)],
        [pltpu.SemaphoreType.DMA for _ in range(NBUF)],
        [pltpu.SemaphoreType.DMA for _ in range(NBUF)],
    ],
)
def _emb(x2_hbm, tab_hbm, pe_hbm, out_hbm, idx_v, pe_v, bufs, gsems, ssems):
    wid = lax.axis_index("s") * NC + lax.axis_index("c")
    p0 = wid * POS_W
    for b in range(B):
        pltpu.sync_copy(
            x2_hbm.at[pl.ds(b * (S // CHUNK) + wid * 2, 2)],
            idx_v.at[pl.ds(b * 2, 2)],
        )
    pltpu.sync_copy(pe_hbm.at[pl.ds(p0, POS_W)], pe_v)

    def gather(t):
        return pltpu.async_copy(
            tab_hbm.at[idx_v.at[t]], bufs[t % NBUF], gsems[t % NBUF]
        )

    gathers = {t: gather(t) for t in range(LOOKAHEAD)}
    stores = {}
    for t in range(NCHUNK):
        m = t % NBUF
        b, c2 = t // 2, t % 2
        if t + LOOKAHEAD < NCHUNK:
            prev = t + LOOKAHEAD - NBUF
            if prev >= 0:
                stores.pop(prev).wait()
            gathers[t + LOOKAHEAD] = gather(t + LOOKAHEAD)
        gathers.pop(t).wait()

        def row_body(i, _, m=m, c2=c2):
            for k in range(D // LANES):
                sl = pl.ds(k * LANES, LANES)
                plsc.addupdate(bufs[m].at[i, sl], pe_v[c2 * CHUNK + i, sl])
            return 0

        lax.fori_loop(0, CHUNK, row_body, 0)
        stores[t] = pltpu.async_copy(
            bufs[m], out_hbm.at[pl.ds(b * S + p0 + c2 * CHUNK, CHUNK)], ssems[m]
        )
    for t in sorted(stores):
        stores.pop(t).wait()


def kernel(x, tok_table, pe):
    x2 = x.reshape(BT // CHUNK, CHUNK)
    out = _emb(x2, tok_table, pe)
    return out.reshape(B, S, D)


# speedup vs baseline: 1.4872x; 1.0329x over previous
import functools

import jax
import jax.numpy as jnp
from jax import lax
from jax.experimental import pallas as pl
from jax.experimental.pallas import tpu as pltpu
from jax.experimental.pallas import tpu_sc as plsc

NC = 2
NS = 16
LANES = 16
NW = NC * NS

B = 4
S = 8192
D = 128
BT = B * S
POS_W = S // NW
CH = 64
NPH = POS_W // CH

_mesh = plsc.VectorSubcoreMesh(
    core_axis_name="c", subcore_axis_name="s", num_cores=NC, num_subcores=NS
)


@functools.partial(
    pl.kernel,
    out_type=jax.ShapeDtypeStruct((BT, D), jnp.float32),
    mesh=_mesh,
    scratch_types=[
        pltpu.VMEM((B * NPH, CH), jnp.int32),
        pltpu.VMEM((POS_W, D), jnp.float32),
        [[pltpu.VMEM((CH, D), jnp.float32) for _ in range(B)] for _ in range(2)],
        [[pltpu.SemaphoreType.DMA for _ in range(B)] for _ in range(2)],
        [[pltpu.SemaphoreType.DMA for _ in range(B)] for _ in range(2)],
    ],
)
def _emb(x64_hbm, tab_hbm, pe_hbm, out_hbm, idx_v, pe_v, bufs, gsems, ssems):
    wid = lax.axis_index("s") * NC + lax.axis_index("c")
    p0 = wid * POS_W
    for b in range(B):
        pltpu.sync_copy(
            x64_hbm.at[pl.ds(b * (S // CH) + wid * NPH, NPH)],
            idx_v.at[pl.ds(b * NPH, NPH)],
        )
    pltpu.sync_copy(pe_hbm.at[pl.ds(p0, POS_W)], pe_v)

    def gather_phase(p):
        st = p % 2
        return [
            pltpu.async_copy(
                tab_hbm.at[idx_v.at[b * NPH + p]], bufs[st][b], gsems[st][b]
            )
            for b in range(B)
        ]

    gathers = {0: gather_phase(0)}
    stores = {}
    for p in range(NPH):
        st = p % 2
        if p + 1 < NPH:
            if p - 1 in stores:
                for h in stores.pop(p - 1):
                    h.wait()
            gathers[p + 1] = gather_phase(p + 1)
        for h in gathers.pop(p):
            h.wait()

        @plsc.parallel_loop(0, CH, step=1, unroll=2)
        def row_body(i, p=p, st=st):
            for k in range(D // LANES):
                sl = pl.ds(k * LANES, LANES)
                v = pe_v[p * CH + i, sl]
                for b in range(B):
                    plsc.addupdate(bufs[st][b].at[i, sl], v)

        stores[p] = [
            pltpu.async_copy(
                bufs[st][b],
                out_hbm.at[pl.ds(b * S + p0 + p * CH, CH)],
                ssems[st][b],
            )
            for b in range(B)
        ]
    for p in sorted(stores):
        for h in stores.pop(p):
            h.wait()


def kernel(x, tok_table, pe):
    x64 = x.reshape(BT // CH, CH)
    out = _emb(x64, tok_table, pe)
    return out.reshape(B, S, D)


# speedup vs baseline: 1.5327x; 1.0306x over previous
import functools

import jax
import jax.numpy as jnp
from jax import lax
from jax.experimental import pallas as pl
from jax.experimental.pallas import tpu as pltpu
from jax.experimental.pallas import tpu_sc as plsc

NC = 2
NS = 16
LANES = 16
NW = NC * NS

B = 4
S = 8192
D = 128
BT = B * S
POS_W = S // NW
CH = 64
NPH = POS_W // CH

_mesh = plsc.VectorSubcoreMesh(
    core_axis_name="c", subcore_axis_name="s", num_cores=NC, num_subcores=NS
)


@functools.partial(
    pl.kernel,
    out_type=jax.ShapeDtypeStruct((BT, D), jnp.float32),
    mesh=_mesh,
    scratch_types=[
        pltpu.VMEM((B * NPH, CH), jnp.int32),
        pltpu.VMEM((POS_W, D), jnp.float32),
        [[pltpu.VMEM((CH, D), jnp.float32) for _ in range(B)] for _ in range(2)],
        [[pltpu.SemaphoreType.DMA for _ in range(B)] for _ in range(2)],
        [[pltpu.SemaphoreType.DMA for _ in range(B)] for _ in range(2)],
        pltpu.SemaphoreType.DMA,
        pltpu.SemaphoreType.DMA,
    ],
)
def _emb(x_hbm, tab_hbm, pe_hbm, out_hbm, idx_v, pe_v, bufs, gsems, ssems,
         isem, psem):
    wid = lax.axis_index("s") * NC + lax.axis_index("c")
    p0 = wid * POS_W
    idx_copies = [
        pltpu.async_copy(
            x_hbm.at[b, pl.ds(p0 + p * CH, CH)],
            idx_v.at[b * NPH + p],
            isem,
        )
        for b in range(B)
        for p in range(NPH)
    ]
    pe_copy = pltpu.async_copy(pe_hbm.at[pl.ds(p0, POS_W)], pe_v, psem)
    for h in idx_copies:
        h.wait()

    def gather_phase(p):
        st = p % 2
        return [
            pltpu.async_copy(
                tab_hbm.at[idx_v.at[b * NPH + p]], bufs[st][b], gsems[st][b]
            )
            for b in range(B)
        ]

    gathers = {0: gather_phase(0)}
    stores = {}
    pe_pending = [pe_copy]
    for p in range(NPH):
        st = p % 2
        if p + 1 < NPH:
            if p - 1 in stores:
                for h in stores.pop(p - 1):
                    h.wait()
            gathers[p + 1] = gather_phase(p + 1)
        for h in gathers.pop(p):
            h.wait()
        if pe_pending:
            pe_pending.pop().wait()

        @plsc.parallel_loop(0, CH, step=1, unroll=2)
        def row_body(i, p=p, st=st):
            for k in range(D // LANES):
                sl = pl.ds(k * LANES, LANES)
                v = pe_v[p * CH + i, sl]
                for b in range(B):
                    plsc.addupdate(bufs[st][b].at[i, sl], v)

        stores[p] = [
            pltpu.async_copy(
                bufs[st][b],
                out_hbm.at[pl.ds(b * S + p0 + p * CH, CH)],
                ssems[st][b],
            )
            for b in range(B)
        ]
    for p in sorted(stores):
        for h in stores.pop(p):
            h.wait()


def kernel(x, tok_table, pe):
    out = _emb(x, tok_table, pe)
    return out.reshape(B, S, D)


# speedup vs baseline: 1.6627x; 1.0848x over previous
import functools

import jax
import jax.numpy as jnp
from jax import lax
from jax.experimental import pallas as pl
from jax.experimental.pallas import tpu as pltpu
from jax.experimental.pallas import tpu_sc as plsc

NC = 2
NS = 16
LANES = 16
NW = NC * NS

B = 4
S = 8192
D = 128
BT = B * S
POS_W = S // NW
CH = 64
NPH = POS_W // CH
NV = D // LANES

_mesh = plsc.VectorSubcoreMesh(
    core_axis_name="c", subcore_axis_name="s", num_cores=NC, num_subcores=NS
)


@functools.partial(
    pl.kernel,
    out_type=jax.ShapeDtypeStruct((BT, D), jnp.float32),
    mesh=_mesh,
    scratch_types=[
        pltpu.VMEM((B * NPH, CH), jnp.int32),
        pltpu.VMEM((2, D), jnp.float32),
        [[pltpu.VMEM((CH, D), jnp.float32) for _ in range(B)] for _ in range(2)],
        [[pltpu.SemaphoreType.DMA for _ in range(B)] for _ in range(2)],
        [[pltpu.SemaphoreType.DMA for _ in range(B)] for _ in range(2)],
        pltpu.SemaphoreType.DMA,
        pltpu.SemaphoreType.DMA,
    ],
)
def _emb(x_hbm, tab_hbm, pe_hbm, out_hbm, idx_v, seed_v, bufs, gsems, ssems,
         isem, psem):
    wid = lax.axis_index("s") * NC + lax.axis_index("c")
    p0 = wid * POS_W
    idx_copies = [
        pltpu.async_copy(
            x_hbm.at[b, pl.ds(p0 + p * CH, CH)],
            idx_v.at[b * NPH + p],
            isem,
        )
        for b in range(B)
        for p in range(NPH)
    ]
    seed_copies = [
        pltpu.async_copy(pe_hbm.at[pl.ds(p0, 1)], seed_v.at[pl.ds(0, 1)], psem),
        pltpu.async_copy(pe_hbm.at[pl.ds(1, 1)], seed_v.at[pl.ds(1, 1)], psem),
    ]
    for h in idx_copies:
        h.wait()

    def gather_phase(p):
        st = p % 2
        return [
            pltpu.async_copy(
                tab_hbm.at[idx_v.at[b * NPH + p]], bufs[st][b], gsems[st][b]
            )
            for b in range(B)
        ]

    gathers = {0: gather_phase(0)}
    for h in seed_copies:
        h.wait()

    iota = lax.iota(jnp.int32, LANES)
    swap = iota ^ 1
    dup_odd = iota | 1
    dup_even = iota - (iota & 1)
    sign = (1 - 2 * (iota & 1)).astype(jnp.float32)
    w1 = [seed_v[1, pl.ds(j * LANES, LANES)][dup_odd] for j in range(NV)]
    w2 = [
        seed_v[1, pl.ds(j * LANES, LANES)][dup_even] * sign for j in range(NV)
    ]
    carry = tuple(seed_v[0, pl.ds(j * LANES, LANES)] for j in range(NV))

    stores = {}
    for p in range(NPH):
        st = p % 2
        if p + 1 < NPH:
            if p - 1 in stores:
                for h in stores.pop(p - 1):
                    h.wait()
            gathers[p + 1] = gather_phase(p + 1)
        for h in gathers.pop(p):
            h.wait()

        @plsc.parallel_loop(0, CH, step=1, unroll=2, carry=carry)
        def row_body(i, pe_regs, st=st):
            for j in range(NV):
                for b in range(B):
                    plsc.addupdate(
                        bufs[st][b].at[i, pl.ds(j * LANES, LANES)], pe_regs[j]
                    )
            return tuple(
                v * w1[j] + v[swap] * w2[j] for j, v in enumerate(pe_regs)
            )

        carry = row_body
        stores[p] = [
            pltpu.async_copy(
                bufs[st][b],
                out_hbm.at[pl.ds(b * S + p0 + p * CH, CH)],
                ssems[st][b],
            )
            for b in range(B)
        ]
    for p in sorted(stores):
        for h in stores.pop(p):
            h.wait()


def kernel(x, tok_table, pe):
    out = _emb(x, tok_table, pe)
    return out.reshape(B, S, D)
